# baseline (speedup 1.0000x reference)
import functools

import jax
import jax.numpy as jnp
from jax import lax
from jax.experimental import pallas as pl
from jax.experimental.pallas import tpu as pltpu
from jax.experimental.pallas import tpu_sc as plsc

_NUM_CORES = 2
_NUM_SUBCORES = 16
_NW = _NUM_CORES * _NUM_SUBCORES
_L = 16
_NUM_INS = 64

_BLOCK_ROWS = 2048


def _sc_hist(ins_hbm, parts_hbm, chunk_v, cnt_v):
    wid = lax.axis_index("s") * _NUM_CORES + lax.axis_index("c")
    chunk = chunk_v.shape[0]
    pltpu.sync_copy(ins_hbm.at[pl.ds(wid * chunk, chunk)], chunk_v)

    zeros = jnp.zeros((_L,), jnp.int32)
    for i in range(_NUM_INS // _L):
        cnt_v[pl.ds(i * _L, _L)] = zeros
    ones = jnp.ones((_L,), jnp.int32)

    def body(i, carry):
        v = chunk_v[pl.ds(i * _L, _L)]
        plsc.addupdate_scatter(cnt_v, [v], ones)
        return carry

    lax.fori_loop(0, chunk // _L, body, 0)
    pltpu.sync_copy(cnt_v, parts_hbm.at[wid])


def _sc_gather(parts_hbm, ins_hbm, d_hbm, parts_v, dil_v, chunk_v, dout_v):
    wid = lax.axis_index("s") * _NUM_CORES + lax.axis_index("c")
    chunk = chunk_v.shape[0]
    pltpu.sync_copy(parts_hbm, parts_v)
    pltpu.sync_copy(ins_hbm.at[pl.ds(wid * chunk, chunk)], chunk_v)

    zeros = jnp.zeros((_L,), jnp.int32)
    for i in range(_NUM_INS // _L):
        acc = zeros
        for r in range(_NW):
            acc = acc + parts_v[r, pl.ds(i * _L, _L)]
        dil = (
            jnp.int32(1)
            + (acc >= 3).astype(jnp.int32)
            + (acc >= 7).astype(jnp.int32)
            + (acc >= 15).astype(jnp.int32)
        )
        dil_v[pl.ds(i * _L, _L)] = dil.astype(jnp.float32)

    def body(i, carry):
        v = chunk_v[pl.ds(i * _L, _L)]
        dout_v[pl.ds(i * _L, _L)] = plsc.load_gather(dil_v, [v])
        return carry

    lax.fori_loop(0, chunk // _L, body, 0)
    pltpu.sync_copy(dout_v, d_hbm.at[pl.ds(wid * chunk, chunk)])


def _make_sc_parts(n):
    chunk = n // _NW
    mesh = plsc.VectorSubcoreMesh(core_axis_name="c", subcore_axis_name="s")
    return functools.partial(
        pl.kernel,
        mesh=mesh,
        out_type=jax.ShapeDtypeStruct((_NW, _NUM_INS), jnp.int32),
        scratch_types=[
            pltpu.VMEM((chunk,), jnp.int32),
            pltpu.VMEM((_NUM_INS,), jnp.int32),
        ],
        compiler_params=pltpu.CompilerParams(needs_layout_passes=False),
    )(_sc_hist)


def _make_sc_dilation(n):
    chunk = n // _NW
    mesh = plsc.VectorSubcoreMesh(core_axis_name="c", subcore_axis_name="s")
    return functools.partial(
        pl.kernel,
        mesh=mesh,
        out_type=jax.ShapeDtypeStruct((n,), jnp.float32),
        scratch_types=[
            pltpu.VMEM((_NW, _NUM_INS), jnp.int32),
            pltpu.VMEM((_NUM_INS,), jnp.float32),
            pltpu.VMEM((chunk,), jnp.int32),
            pltpu.VMEM((chunk,), jnp.float32),
        ],
        compiler_params=pltpu.CompilerParams(needs_layout_passes=False),
    )(_sc_gather)


_COPY_LAG = 4


def _exp_body(x_ref, feat_hbm, out_ref, fout_hbm, copy_sem):
    f32 = jnp.float32
    i32 = jnp.int32
    bf16 = jnp.bfloat16
    b = x_ref.shape[1]
    i = pl.program_id(0)
    n_i = pl.num_programs(0)
    rows = feat_hbm.shape[0] // n_i

    def copy_chunk(j):
        return pltpu.make_async_copy(
            feat_hbm.at[pl.ds(j * rows, rows)],
            fout_hbm.at[pl.ds(j * rows, rows)],
            copy_sem,
        )

    copy_chunk(i).start()

    @pl.when(i >= _COPY_LAG)
    def _():
        copy_chunk(i - _COPY_LAG).wait()
    cc = lax.broadcasted_iota(i32, (108, 9), 0)
    jj = lax.broadcasted_iota(i32, (108, 9), 1)
    j = cc & 3
    k = cc >> 2
    dz = k // 9 - 1
    dy = (k // 3) % 3 - 1
    dx = k % 3 - 1
    offv = jnp.where(j == 0, 0, jnp.where(j == 1, dz, jnp.where(j == 2, dy, dx)))
    sel_hi = jnp.where(j == (jj & 3), 8, 0)
    sel_lo = (j == (jj & 3)).astype(i32)
    m = jnp.where(jj < 4, sel_hi, jnp.where(jj < 8, sel_lo, offv)).astype(bf16)
    outf = jnp.dot(m, x_ref[...], preferred_element_type=f32)
    out_ref[...] = outf.astype(i32).reshape(27, 4, b)

    @pl.when(i == n_i - 1)
    def _():
        for _t in range(_COPY_LAG):
            copy_chunk(0).wait()


def kernel(features, indices, ins_indices_batch, ins_ids):
    n = features.shape[0]
    parts = _make_sc_parts(n)(ins_indices_batch)
    d = _make_sc_dilation(n)(parts, ins_indices_batch)
    idx_t = indices.T
    x = jnp.concatenate(
        [
            (idx_t >> 3).astype(jnp.bfloat16),
            (idx_t & 7).astype(jnp.bfloat16),
            d.reshape(1, n).astype(jnp.bfloat16),
        ],
        axis=0,
    )
    b = _BLOCK_ROWS
    out3, feat_out = pl.pallas_call(
        _exp_body,
        grid=(n // b,),
        in_specs=[
            pl.BlockSpec((9, b), lambda i: (0, i)),
            pl.BlockSpec(memory_space=pl.ANY),
        ],
        out_specs=(
            pl.BlockSpec((27, 4, b), lambda i: (0, 0, i)),
            pl.BlockSpec(memory_space=pl.ANY),
        ),
        out_shape=(
            jax.ShapeDtypeStruct((27, 4, n), jnp.int32),
            jax.ShapeDtypeStruct(features.shape, features.dtype),
        ),
        scratch_shapes=[pltpu.SemaphoreType.DMA],
    )(x, features)
    return feat_out, jnp.transpose(out3, (2, 0, 1))


# speedup vs baseline: 19.5693x; 19.5693x over previous
import functools

import jax
import jax.numpy as jnp
from jax import lax
from jax.experimental import pallas as pl
from jax.experimental.pallas import tpu as pltpu
from jax.experimental.pallas import tpu_sc as plsc

_NUM_CORES = 2
_NUM_SUBCORES = 16
_NW = _NUM_CORES * _NUM_SUBCORES
_L = 16
_NUM_INS = 64

_BLOCK_ROWS = 2048


def _sc_hist(ins_hbm, parts_hbm, chunk_v, cnt_v):
    wid = lax.axis_index("s") * _NUM_CORES + lax.axis_index("c")
    chunk = chunk_v.shape[0]
    pltpu.sync_copy(ins_hbm.at[pl.ds(wid * chunk, chunk)], chunk_v)

    zeros = jnp.zeros((_L,), jnp.int32)
    for i in range(_NUM_INS // _L):
        cnt_v[pl.ds(i * _L, _L)] = zeros
    ones = jnp.ones((_L,), jnp.int32)

    def body(i, carry):
        v = chunk_v[pl.ds(i * _L, _L)]
        plsc.addupdate_scatter(cnt_v, [v], ones)
        return carry

    lax.fori_loop(0, chunk // _L, body, 0)
    pltpu.sync_copy(cnt_v, parts_hbm.at[wid])


def _sc_gather(parts_hbm, ins_hbm, d_hbm, parts_v, dil_v, chunk_v, dout_v):
    wid = lax.axis_index("s") * _NUM_CORES + lax.axis_index("c")
    chunk = chunk_v.shape[0]
    pltpu.sync_copy(parts_hbm, parts_v)
    pltpu.sync_copy(ins_hbm.at[pl.ds(wid * chunk, chunk)], chunk_v)

    zeros = jnp.zeros((_L,), jnp.int32)
    for i in range(_NUM_INS // _L):
        acc = zeros
        for r in range(_NW):
            acc = acc + parts_v[r, pl.ds(i * _L, _L)]
        dil = (
            jnp.int32(1)
            + (acc >= 3).astype(jnp.int32)
            + (acc >= 7).astype(jnp.int32)
            + (acc >= 15).astype(jnp.int32)
        )
        dil_v[pl.ds(i * _L, _L)] = dil.astype(jnp.float32)

    def body(i, carry):
        v = chunk_v[pl.ds(i * _L, _L)]
        dout_v[pl.ds(i * _L, _L)] = plsc.load_gather(dil_v, [v])
        return carry

    lax.fori_loop(0, chunk // _L, body, 0)
    pltpu.sync_copy(dout_v, d_hbm.at[pl.ds(wid * chunk, chunk)])


def _make_sc_parts(n):
    chunk = n // _NW
    mesh = plsc.VectorSubcoreMesh(core_axis_name="c", subcore_axis_name="s")
    return functools.partial(
        pl.kernel,
        mesh=mesh,
        out_type=jax.ShapeDtypeStruct((_NW, _NUM_INS), jnp.int32),
        scratch_types=[
            pltpu.VMEM((chunk,), jnp.int32),
            pltpu.VMEM((_NUM_INS,), jnp.int32),
        ],
        compiler_params=pltpu.CompilerParams(needs_layout_passes=False),
    )(_sc_hist)


def _make_sc_dilation(n):
    chunk = n // _NW
    mesh = plsc.VectorSubcoreMesh(core_axis_name="c", subcore_axis_name="s")
    return functools.partial(
        pl.kernel,
        mesh=mesh,
        out_type=jax.ShapeDtypeStruct((n,), jnp.float32),
        scratch_types=[
            pltpu.VMEM((_NW, _NUM_INS), jnp.int32),
            pltpu.VMEM((_NUM_INS,), jnp.float32),
            pltpu.VMEM((chunk,), jnp.int32),
            pltpu.VMEM((chunk,), jnp.float32),
        ],
        compiler_params=pltpu.CompilerParams(needs_layout_passes=False),
    )(_sc_gather)


def _exp_body(x_ref, feat_ref, out_ref, fout_ref):
    f32 = jnp.float32
    i32 = jnp.int32
    bf16 = jnp.bfloat16
    b = x_ref.shape[1]
    fout_ref[...] = feat_ref[...]
    cc = lax.broadcasted_iota(i32, (108, 9), 0)
    jj = lax.broadcasted_iota(i32, (108, 9), 1)
    j = cc & 3
    k = cc >> 2
    dz = k // 9 - 1
    dy = (k // 3) % 3 - 1
    dx = k % 3 - 1
    offv = jnp.where(j == 0, 0, jnp.where(j == 1, dz, jnp.where(j == 2, dy, dx)))
    sel_hi = jnp.where(j == (jj & 3), 8, 0)
    sel_lo = (j == (jj & 3)).astype(i32)
    m = jnp.where(jj < 4, sel_hi, jnp.where(jj < 8, sel_lo, offv)).astype(bf16)
    outf = jnp.dot(m, x_ref[...], preferred_element_type=f32)
    out_ref[...] = outf.astype(i32).reshape(27, 4, b)


def kernel(features, indices, ins_indices_batch, ins_ids):
    n = features.shape[0]
    parts = _make_sc_parts(n)(ins_indices_batch)
    d = _make_sc_dilation(n)(parts, ins_indices_batch)
    idx_t = indices.T
    x = jnp.concatenate(
        [
            (idx_t >> 3).astype(jnp.bfloat16),
            (idx_t & 7).astype(jnp.bfloat16),
            d.reshape(1, n).astype(jnp.bfloat16),
        ],
        axis=0,
    )
    b = _BLOCK_ROWS
    out3, feat_out = pl.pallas_call(
        _exp_body,
        grid=(n // b,),
        in_specs=[
            pl.BlockSpec((9, b), lambda i: (0, i)),
            pl.BlockSpec((b, features.shape[1]), lambda i: (i, 0)),
        ],
        out_specs=(
            pl.BlockSpec((27, 4, b), lambda i: (0, 0, i)),
            pl.BlockSpec((b, features.shape[1]), lambda i: (i, 0)),
        ),
        out_shape=(
            jax.ShapeDtypeStruct((27, 4, n), jnp.int32),
            jax.ShapeDtypeStruct(features.shape, features.dtype),
        ),
    )(x, features)
    return feat_out, jnp.transpose(out3, (2, 0, 1))


# speedup vs baseline: 23.5612x; 1.2040x over previous
import functools

import jax
import jax.numpy as jnp
from jax import lax
from jax.experimental import pallas as pl
from jax.experimental.pallas import tpu as pltpu
from jax.experimental.pallas import tpu_sc as plsc

_NUM_CORES = 2
_NUM_SUBCORES = 16
_NW = _NUM_CORES * _NUM_SUBCORES
_L = 16
_NUM_INS = 64

_BLOCK_ROWS = 4096


def _sc_hist(ins_hbm, parts_hbm, chunk_v, cnt_v):
    wid = lax.axis_index("s") * _NUM_CORES + lax.axis_index("c")
    chunk = chunk_v.shape[0]
    pltpu.sync_copy(ins_hbm.at[pl.ds(wid * chunk, chunk)], chunk_v)

    zeros = jnp.zeros((_L,), jnp.int32)
    for i in range(_NUM_INS // _L):
        cnt_v[pl.ds(i * _L, _L)] = zeros
    ones = jnp.ones((_L,), jnp.int32)

    def body(i, carry):
        v = chunk_v[pl.ds(i * _L, _L)]
        plsc.addupdate_scatter(cnt_v, [v], ones)
        return carry

    lax.fori_loop(0, chunk // _L, body, 0)
    pltpu.sync_copy(cnt_v, parts_hbm.at[wid])


def _sc_gather(parts_hbm, ins_hbm, d_hbm, parts_v, dil_v, chunk_v, dout_v):
    wid = lax.axis_index("s") * _NUM_CORES + lax.axis_index("c")
    chunk = chunk_v.shape[0]
    pltpu.sync_copy(parts_hbm, parts_v)
    pltpu.sync_copy(ins_hbm.at[pl.ds(wid * chunk, chunk)], chunk_v)

    zeros = jnp.zeros((_L,), jnp.int32)
    for i in range(_NUM_INS // _L):
        acc = zeros
        for r in range(_NW):
            acc = acc + parts_v[r, pl.ds(i * _L, _L)]
        dil = (
            jnp.int32(1)
            + (acc >= 3).astype(jnp.int32)
            + (acc >= 7).astype(jnp.int32)
            + (acc >= 15).astype(jnp.int32)
        )
        dil_v[pl.ds(i * _L, _L)] = dil.astype(jnp.float32)

    def body(i, carry):
        v = chunk_v[pl.ds(i * _L, _L)]
        dout_v[pl.ds(i * _L, _L)] = plsc.load_gather(dil_v, [v])
        return carry

    lax.fori_loop(0, chunk // _L, body, 0)
    pltpu.sync_copy(dout_v, d_hbm.at[pl.ds(wid * chunk, chunk)])


def _make_sc_parts(n):
    chunk = n // _NW
    mesh = plsc.VectorSubcoreMesh(core_axis_name="c", subcore_axis_name="s")
    return functools.partial(
        pl.kernel,
        mesh=mesh,
        out_type=jax.ShapeDtypeStruct((_NW, _NUM_INS), jnp.int32),
        scratch_types=[
            pltpu.VMEM((chunk,), jnp.int32),
            pltpu.VMEM((_NUM_INS,), jnp.int32),
        ],
        compiler_params=pltpu.CompilerParams(needs_layout_passes=False),
    )(_sc_hist)


def _make_sc_dilation(n):
    chunk = n // _NW
    mesh = plsc.VectorSubcoreMesh(core_axis_name="c", subcore_axis_name="s")
    return functools.partial(
        pl.kernel,
        mesh=mesh,
        out_type=jax.ShapeDtypeStruct((n,), jnp.float32),
        scratch_types=[
            pltpu.VMEM((_NW, _NUM_INS), jnp.int32),
            pltpu.VMEM((_NUM_INS,), jnp.float32),
            pltpu.VMEM((chunk,), jnp.int32),
            pltpu.VMEM((chunk,), jnp.float32),
        ],
        compiler_params=pltpu.CompilerParams(needs_layout_passes=False),
    )(_sc_gather)


def _exp_body(x_ref, feat_ref, out_ref, fout_ref):
    f32 = jnp.float32
    i32 = jnp.int32
    bf16 = jnp.bfloat16
    b = x_ref.shape[1]
    fout_ref[...] = feat_ref[...]
    cc = lax.broadcasted_iota(i32, (108, 9), 0)
    jj = lax.broadcasted_iota(i32, (108, 9), 1)
    j = cc & 3
    k = cc >> 2
    dz = k // 9 - 1
    dy = (k // 3) % 3 - 1
    dx = k % 3 - 1
    offv = jnp.where(j == 0, 0, jnp.where(j == 1, dz, jnp.where(j == 2, dy, dx)))
    sel_hi = jnp.where(j == (jj & 3), 8, 0)
    sel_lo = (j == (jj & 3)).astype(i32)
    m = jnp.where(jj < 4, sel_hi, jnp.where(jj < 8, sel_lo, offv)).astype(bf16)
    outf = jnp.dot(m, x_ref[...], preferred_element_type=f32)
    out_ref[...] = outf.astype(i32).reshape(27, 4, b)


def kernel(features, indices, ins_indices_batch, ins_ids):
    n = features.shape[0]
    parts = _make_sc_parts(n)(ins_indices_batch)
    d = _make_sc_dilation(n)(parts, ins_indices_batch)
    idx_t = indices.T
    x = jnp.concatenate(
        [
            (idx_t >> 3).astype(jnp.bfloat16),
            (idx_t & 7).astype(jnp.bfloat16),
            d.reshape(1, n).astype(jnp.bfloat16),
        ],
        axis=0,
    )
    b = _BLOCK_ROWS
    out3, feat_out = pl.pallas_call(
        _exp_body,
        grid=(n // b,),
        in_specs=[
            pl.BlockSpec((9, b), lambda i: (0, i)),
            pl.BlockSpec((b, features.shape[1]), lambda i: (i, 0)),
        ],
        out_specs=(
            pl.BlockSpec((27, 4, b), lambda i: (0, 0, i)),
            pl.BlockSpec((b, features.shape[1]), lambda i: (i, 0)),
        ),
        out_shape=(
            jax.ShapeDtypeStruct((27, 4, n), jnp.int32),
            jax.ShapeDtypeStruct(features.shape, features.dtype),
        ),
    )(x, features)
    return feat_out, jnp.transpose(out3, (2, 0, 1))


# speedup vs baseline: 24.7364x; 1.0499x over previous
import functools

import jax
import jax.numpy as jnp
from jax import lax
from jax.experimental import pallas as pl
from jax.experimental.pallas import tpu as pltpu
from jax.experimental.pallas import tpu_sc as plsc

_NUM_CORES = 2
_NUM_SUBCORES = 16
_NW = _NUM_CORES * _NUM_SUBCORES
_L = 16
_NUM_INS = 64

_BLOCK_ROWS = 8192


def _sc_hist(ins_hbm, parts_hbm, chunk_v, cnt_v):
    wid = lax.axis_index("s") * _NUM_CORES + lax.axis_index("c")
    chunk = chunk_v.shape[0]
    pltpu.sync_copy(ins_hbm.at[pl.ds(wid * chunk, chunk)], chunk_v)

    zeros = jnp.zeros((_L,), jnp.int32)
    for i in range(_NUM_INS // _L):
        cnt_v[pl.ds(i * _L, _L)] = zeros
    ones = jnp.ones((_L,), jnp.int32)

    def body(i, carry):
        v = chunk_v[pl.ds(i * _L, _L)]
        plsc.addupdate_scatter(cnt_v, [v], ones)
        return carry

    lax.fori_loop(0, chunk // _L, body, 0)
    pltpu.sync_copy(cnt_v, parts_hbm.at[wid])


def _sc_gather(parts_hbm, ins_hbm, d_hbm, parts_v, dil_v, chunk_v, dout_v):
    wid = lax.axis_index("s") * _NUM_CORES + lax.axis_index("c")
    chunk = chunk_v.shape[0]
    pltpu.sync_copy(parts_hbm, parts_v)
    pltpu.sync_copy(ins_hbm.at[pl.ds(wid * chunk, chunk)], chunk_v)

    zeros = jnp.zeros((_L,), jnp.int32)
    for i in range(_NUM_INS // _L):
        acc = zeros
        for r in range(_NW):
            acc = acc + parts_v[r, pl.ds(i * _L, _L)]
        dil = (
            jnp.int32(1)
            + (acc >= 3).astype(jnp.int32)
            + (acc >= 7).astype(jnp.int32)
            + (acc >= 15).astype(jnp.int32)
        )
        dil_v[pl.ds(i * _L, _L)] = dil.astype(jnp.float32)

    def body(i, carry):
        v = chunk_v[pl.ds(i * _L, _L)]
        dout_v[pl.ds(i * _L, _L)] = plsc.load_gather(dil_v, [v])
        return carry

    lax.fori_loop(0, chunk // _L, body, 0)
    pltpu.sync_copy(dout_v, d_hbm.at[pl.ds(wid * chunk, chunk)])


def _make_sc_parts(n):
    chunk = n // _NW
    mesh = plsc.VectorSubcoreMesh(core_axis_name="c", subcore_axis_name="s")
    return functools.partial(
        pl.kernel,
        mesh=mesh,
        out_type=jax.ShapeDtypeStruct((_NW, _NUM_INS), jnp.int32),
        scratch_types=[
            pltpu.VMEM((chunk,), jnp.int32),
            pltpu.VMEM((_NUM_INS,), jnp.int32),
        ],
        compiler_params=pltpu.CompilerParams(needs_layout_passes=False),
    )(_sc_hist)


def _make_sc_dilation(n):
    chunk = n // _NW
    mesh = plsc.VectorSubcoreMesh(core_axis_name="c", subcore_axis_name="s")
    return functools.partial(
        pl.kernel,
        mesh=mesh,
        out_type=jax.ShapeDtypeStruct((n,), jnp.float32),
        scratch_types=[
            pltpu.VMEM((_NW, _NUM_INS), jnp.int32),
            pltpu.VMEM((_NUM_INS,), jnp.float32),
            pltpu.VMEM((chunk,), jnp.int32),
            pltpu.VMEM((chunk,), jnp.float32),
        ],
        compiler_params=pltpu.CompilerParams(needs_layout_passes=False),
    )(_sc_gather)


def _exp_body(x_ref, feat_ref, out_ref, fout_ref):
    f32 = jnp.float32
    i32 = jnp.int32
    bf16 = jnp.bfloat16
    b = x_ref.shape[1]
    fout_ref[...] = feat_ref[...]
    cc = lax.broadcasted_iota(i32, (108, 9), 0)
    jj = lax.broadcasted_iota(i32, (108, 9), 1)
    j = cc & 3
    k = cc >> 2
    dz = k // 9 - 1
    dy = (k // 3) % 3 - 1
    dx = k % 3 - 1
    offv = jnp.where(j == 0, 0, jnp.where(j == 1, dz, jnp.where(j == 2, dy, dx)))
    sel_hi = jnp.where(j == (jj & 3), 8, 0)
    sel_lo = (j == (jj & 3)).astype(i32)
    m = jnp.where(jj < 4, sel_hi, jnp.where(jj < 8, sel_lo, offv)).astype(bf16)
    outf = jnp.dot(m, x_ref[...], preferred_element_type=f32)
    out_ref[...] = outf.astype(i32).reshape(27, 4, b)


def kernel(features, indices, ins_indices_batch, ins_ids):
    n = features.shape[0]
    parts = _make_sc_parts(n)(ins_indices_batch)
    d = _make_sc_dilation(n)(parts, ins_indices_batch)
    idx_t = indices.T
    x = jnp.concatenate(
        [
            (idx_t >> 3).astype(jnp.bfloat16),
            (idx_t & 7).astype(jnp.bfloat16),
            d.reshape(1, n).astype(jnp.bfloat16),
        ],
        axis=0,
    )
    b = _BLOCK_ROWS
    out3, feat_out = pl.pallas_call(
        _exp_body,
        grid=(n // b,),
        in_specs=[
            pl.BlockSpec((9, b), lambda i: (0, i)),
            pl.BlockSpec((b, features.shape[1]), lambda i: (i, 0)),
        ],
        out_specs=(
            pl.BlockSpec((27, 4, b), lambda i: (0, 0, i)),
            pl.BlockSpec((b, features.shape[1]), lambda i: (i, 0)),
        ),
        out_shape=(
            jax.ShapeDtypeStruct((27, 4, n), jnp.int32),
            jax.ShapeDtypeStruct(features.shape, features.dtype),
        ),
    )(x, features)
    return feat_out, jnp.transpose(out3, (2, 0, 1))


# speedup vs baseline: 25.3070x; 1.0231x over previous
import functools

import jax
import jax.numpy as jnp
from jax import lax
from jax.experimental import pallas as pl
from jax.experimental.pallas import tpu as pltpu
from jax.experimental.pallas import tpu_sc as plsc

_NUM_CORES = 2
_NUM_SUBCORES = 16
_NW = _NUM_CORES * _NUM_SUBCORES
_L = 16
_NUM_INS = 64

_BLOCK_ROWS = 16384


def _sc_hist(ins_hbm, parts_hbm, chunk_v, cnt_v):
    wid = lax.axis_index("s") * _NUM_CORES + lax.axis_index("c")
    chunk = chunk_v.shape[0]
    pltpu.sync_copy(ins_hbm.at[pl.ds(wid * chunk, chunk)], chunk_v)

    zeros = jnp.zeros((_L,), jnp.int32)
    for i in range(_NUM_INS // _L):
        cnt_v[pl.ds(i * _L, _L)] = zeros
    ones = jnp.ones((_L,), jnp.int32)

    def body(i, carry):
        v = chunk_v[pl.ds(i * _L, _L)]
        plsc.addupdate_scatter(cnt_v, [v], ones)
        return carry

    lax.fori_loop(0, chunk // _L, body, 0)
    pltpu.sync_copy(cnt_v, parts_hbm.at[wid])


def _sc_gather(parts_hbm, ins_hbm, d_hbm, parts_v, dil_v, chunk_v, dout_v):
    wid = lax.axis_index("s") * _NUM_CORES + lax.axis_index("c")
    chunk = chunk_v.shape[0]
    pltpu.sync_copy(parts_hbm, parts_v)
    pltpu.sync_copy(ins_hbm.at[pl.ds(wid * chunk, chunk)], chunk_v)

    zeros = jnp.zeros((_L,), jnp.int32)
    for i in range(_NUM_INS // _L):
        acc = zeros
        for r in range(_NW):
            acc = acc + parts_v[r, pl.ds(i * _L, _L)]
        dil = (
            jnp.int32(1)
            + (acc >= 3).astype(jnp.int32)
            + (acc >= 7).astype(jnp.int32)
            + (acc >= 15).astype(jnp.int32)
        )
        dil_v[pl.ds(i * _L, _L)] = dil.astype(jnp.float32)

    def body(i, carry):
        v = chunk_v[pl.ds(i * _L, _L)]
        dout_v[pl.ds(i * _L, _L)] = plsc.load_gather(dil_v, [v])
        return carry

    lax.fori_loop(0, chunk // _L, body, 0)
    pltpu.sync_copy(dout_v, d_hbm.at[pl.ds(wid * chunk, chunk)])


def _make_sc_parts(n):
    chunk = n // _NW
    mesh = plsc.VectorSubcoreMesh(core_axis_name="c", subcore_axis_name="s")
    return functools.partial(
        pl.kernel,
        mesh=mesh,
        out_type=jax.ShapeDtypeStruct((_NW, _NUM_INS), jnp.int32),
        scratch_types=[
            pltpu.VMEM((chunk,), jnp.int32),
            pltpu.VMEM((_NUM_INS,), jnp.int32),
        ],
        compiler_params=pltpu.CompilerParams(needs_layout_passes=False),
    )(_sc_hist)


def _make_sc_dilation(n):
    chunk = n // _NW
    mesh = plsc.VectorSubcoreMesh(core_axis_name="c", subcore_axis_name="s")
    return functools.partial(
        pl.kernel,
        mesh=mesh,
        out_type=jax.ShapeDtypeStruct((n,), jnp.float32),
        scratch_types=[
            pltpu.VMEM((_NW, _NUM_INS), jnp.int32),
            pltpu.VMEM((_NUM_INS,), jnp.float32),
            pltpu.VMEM((chunk,), jnp.int32),
            pltpu.VMEM((chunk,), jnp.float32),
        ],
        compiler_params=pltpu.CompilerParams(needs_layout_passes=False),
    )(_sc_gather)


def _exp_body(x_ref, feat_ref, out_ref, fout_ref):
    f32 = jnp.float32
    i32 = jnp.int32
    bf16 = jnp.bfloat16
    b = x_ref.shape[1]
    fout_ref[...] = feat_ref[...]
    cc = lax.broadcasted_iota(i32, (108, 9), 0)
    jj = lax.broadcasted_iota(i32, (108, 9), 1)
    j = cc & 3
    k = cc >> 2
    dz = k // 9 - 1
    dy = (k // 3) % 3 - 1
    dx = k % 3 - 1
    offv = jnp.where(j == 0, 0, jnp.where(j == 1, dz, jnp.where(j == 2, dy, dx)))
    sel_hi = jnp.where(j == (jj & 3), 8, 0)
    sel_lo = (j == (jj & 3)).astype(i32)
    m = jnp.where(jj < 4, sel_hi, jnp.where(jj < 8, sel_lo, offv)).astype(bf16)
    outf = jnp.dot(m, x_ref[...], preferred_element_type=f32)
    out_ref[...] = outf.astype(i32).reshape(27, 4, b)


def kernel(features, indices, ins_indices_batch, ins_ids):
    n = features.shape[0]
    parts = _make_sc_parts(n)(ins_indices_batch)
    d = _make_sc_dilation(n)(parts, ins_indices_batch)
    idx_t = indices.T
    x = jnp.concatenate(
        [
            (idx_t >> 3).astype(jnp.bfloat16),
            (idx_t & 7).astype(jnp.bfloat16),
            d.reshape(1, n).astype(jnp.bfloat16),
        ],
        axis=0,
    )
    b = _BLOCK_ROWS
    out3, feat_out = pl.pallas_call(
        _exp_body,
        grid=(n // b,),
        in_specs=[
            pl.BlockSpec((9, b), lambda i: (0, i)),
            pl.BlockSpec((b, features.shape[1]), lambda i: (i, 0)),
        ],
        out_specs=(
            pl.BlockSpec((27, 4, b), lambda i: (0, 0, i)),
            pl.BlockSpec((b, features.shape[1]), lambda i: (i, 0)),
        ),
        out_shape=(
            jax.ShapeDtypeStruct((27, 4, n), jnp.int32),
            jax.ShapeDtypeStruct(features.shape, features.dtype),
        ),
    )(x, features)
    return feat_out, jnp.transpose(out3, (2, 0, 1))


# speedup vs baseline: 25.4483x; 1.0056x over previous
import functools

import jax
import jax.numpy as jnp
from jax import lax
from jax.experimental import pallas as pl
from jax.experimental.pallas import tpu as pltpu
from jax.experimental.pallas import tpu_sc as plsc

_NUM_CORES = 2
_NUM_SUBCORES = 16
_NW = _NUM_CORES * _NUM_SUBCORES
_L = 16
_NUM_INS = 64

_BLOCK_ROWS = 32768


def _sc_hist(ins_hbm, parts_hbm, chunk_v, cnt_v):
    wid = lax.axis_index("s") * _NUM_CORES + lax.axis_index("c")
    chunk = chunk_v.shape[0]
    pltpu.sync_copy(ins_hbm.at[pl.ds(wid * chunk, chunk)], chunk_v)

    zeros = jnp.zeros((_L,), jnp.int32)
    for i in range(_NUM_INS // _L):
        cnt_v[pl.ds(i * _L, _L)] = zeros
    ones = jnp.ones((_L,), jnp.int32)

    def body(i, carry):
        v = chunk_v[pl.ds(i * _L, _L)]
        plsc.addupdate_scatter(cnt_v, [v], ones)
        return carry

    lax.fori_loop(0, chunk // _L, body, 0)
    pltpu.sync_copy(cnt_v, parts_hbm.at[wid])


def _sc_gather(parts_hbm, ins_hbm, d_hbm, parts_v, dil_v, chunk_v, dout_v):
    wid = lax.axis_index("s") * _NUM_CORES + lax.axis_index("c")
    chunk = chunk_v.shape[0]
    pltpu.sync_copy(parts_hbm, parts_v)
    pltpu.sync_copy(ins_hbm.at[pl.ds(wid * chunk, chunk)], chunk_v)

    zeros = jnp.zeros((_L,), jnp.int32)
    for i in range(_NUM_INS // _L):
        acc = zeros
        for r in range(_NW):
            acc = acc + parts_v[r, pl.ds(i * _L, _L)]
        dil = (
            jnp.int32(1)
            + (acc >= 3).astype(jnp.int32)
            + (acc >= 7).astype(jnp.int32)
            + (acc >= 15).astype(jnp.int32)
        )
        dil_v[pl.ds(i * _L, _L)] = dil.astype(jnp.float32)

    def body(i, carry):
        v = chunk_v[pl.ds(i * _L, _L)]
        dout_v[pl.ds(i * _L, _L)] = plsc.load_gather(dil_v, [v])
        return carry

    lax.fori_loop(0, chunk // _L, body, 0)
    pltpu.sync_copy(dout_v, d_hbm.at[pl.ds(wid * chunk, chunk)])


def _make_sc_parts(n):
    chunk = n // _NW
    mesh = plsc.VectorSubcoreMesh(core_axis_name="c", subcore_axis_name="s")
    return functools.partial(
        pl.kernel,
        mesh=mesh,
        out_type=jax.ShapeDtypeStruct((_NW, _NUM_INS), jnp.int32),
        scratch_types=[
            pltpu.VMEM((chunk,), jnp.int32),
            pltpu.VMEM((_NUM_INS,), jnp.int32),
        ],
        compiler_params=pltpu.CompilerParams(needs_layout_passes=False),
    )(_sc_hist)


def _make_sc_dilation(n):
    chunk = n // _NW
    mesh = plsc.VectorSubcoreMesh(core_axis_name="c", subcore_axis_name="s")
    return functools.partial(
        pl.kernel,
        mesh=mesh,
        out_type=jax.ShapeDtypeStruct((n,), jnp.float32),
        scratch_types=[
            pltpu.VMEM((_NW, _NUM_INS), jnp.int32),
            pltpu.VMEM((_NUM_INS,), jnp.float32),
            pltpu.VMEM((chunk,), jnp.int32),
            pltpu.VMEM((chunk,), jnp.float32),
        ],
        compiler_params=pltpu.CompilerParams(needs_layout_passes=False),
    )(_sc_gather)


def _copy_body(feat_ref, fout_ref):
    fout_ref[...] = feat_ref[...]


def _exp_body(x_ref, out_ref):
    f32 = jnp.float32
    i32 = jnp.int32
    bf16 = jnp.bfloat16
    b = x_ref.shape[1]
    cc = lax.broadcasted_iota(i32, (108, 9), 0)
    jj = lax.broadcasted_iota(i32, (108, 9), 1)
    j = cc & 3
    k = cc >> 2
    dz = k // 9 - 1
    dy = (k // 3) % 3 - 1
    dx = k % 3 - 1
    offv = jnp.where(j == 0, 0, jnp.where(j == 1, dz, jnp.where(j == 2, dy, dx)))
    sel_hi = jnp.where(j == (jj & 3), 8, 0)
    sel_lo = (j == (jj & 3)).astype(i32)
    m = jnp.where(jj < 4, sel_hi, jnp.where(jj < 8, sel_lo, offv)).astype(bf16)
    outf = jnp.dot(m, x_ref[...], preferred_element_type=f32)
    out_ref[...] = outf.astype(i32).reshape(27, 4, b)


def kernel(features, indices, ins_indices_batch, ins_ids):
    n = features.shape[0]
    assert ins_ids.shape[0] == _NUM_INS and n % (_NW * _L) == 0
    bf = 16384
    feat_out = pl.pallas_call(
        _copy_body,
        grid=(n // bf,),
        in_specs=[pl.BlockSpec((bf, features.shape[1]), lambda i: (i, 0))],
        out_specs=pl.BlockSpec((bf, features.shape[1]), lambda i: (i, 0)),
        out_shape=jax.ShapeDtypeStruct(features.shape, features.dtype),
    )(features)
    parts = _make_sc_parts(n)(ins_indices_batch)
    d = _make_sc_dilation(n)(parts, ins_indices_batch)
    idx_t = indices.T
    x = jnp.concatenate(
        [
            (idx_t >> 3).astype(jnp.bfloat16),
            (idx_t & 7).astype(jnp.bfloat16),
            d.reshape(1, n).astype(jnp.bfloat16),
        ],
        axis=0,
    )
    b = _BLOCK_ROWS
    out3 = pl.pallas_call(
        _exp_body,
        grid=(n // b,),
        in_specs=[pl.BlockSpec((9, b), lambda i: (0, i))],
        out_specs=pl.BlockSpec((27, 4, b), lambda i: (0, 0, i)),
        out_shape=jax.ShapeDtypeStruct((27, 4, n), jnp.int32),
    )(x)
    return feat_out, jnp.transpose(out3, (2, 0, 1))
